# initial kernel scaffold (unmeasured)
import jax
import jax.numpy as jnp
from jax import lax
from jax.experimental import pallas as pl
from jax.experimental.pallas import tpu as pltpu

N_DEV = 4
L = 32


def kernel(x, A, B, C):
    Bsz, Sl, D = x.shape
    N = A.shape[1]
    NC = Sl // L

    AT = A.T
    x_e = x[:, :, None, :]
    B_e = B[:, :, :, None]
    C_e = C[:, :, :, None]

    def body(xe_ref, at_ref, be_ref, ce_ref, out_ref,
             hloc_ref, hin_ref, send_sem, recv_sem):
        my = lax.axis_index("i")

        barrier = pltpu.get_barrier_semaphore()
        for k in range(1, N_DEV):
            pl.semaphore_signal(
                barrier, inc=1,
                device_id=((my + k) % N_DEV,),
                device_id_type=pl.DeviceIdType.MESH,
            )
        pl.semaphore_wait(barrier, N_DEV - 1)

        at = at_ref[:, :]
        tau = lax.broadcasted_iota(jnp.float32, (L, N, D), 0)
        powA = jnp.exp(tau * at[None])
        powAinv = jnp.exp(-tau * at[None])
        dA1 = jnp.exp(at)

        h = jnp.zeros((Bsz, N, D), jnp.float32)
        for c in range(NC):
            sl = pl.ds(c * L, L)
            xb = xe_ref[:, sl, :, :] * be_ref[:, sl, :, :]
            q = xb * powAinv[None]
            csum = q
            sh = 1
            while sh < L:
                csum = csum + jnp.concatenate(
                    [jnp.zeros((Bsz, sh, N, D), jnp.float32),
                     csum[:, : L - sh]], axis=1)
                sh *= 2
            h_all = powA[None] * (csum + (dA1[None] * h)[:, None])
            out_ref[:, sl, :] = (h_all * ce_ref[:, sl, :, :]).sum(
                axis=2, keepdims=False)
            h = h_all[:, L - 1]
        hloc_ref[...] = h

        @pl.when(my < N_DEV - 1)
        def _send():
            rdma = pltpu.make_async_remote_copy(
                src_ref=hloc_ref, dst_ref=hin_ref,
                send_sem=send_sem, recv_sem=recv_sem,
                device_id=((my + 1) % N_DEV,),
                device_id_type=pl.DeviceIdType.MESH,
            )
            rdma.start()
            rdma.wait_send()

        @pl.when(my > 0)
        def _recv_and_correct():
            rdma = pltpu.make_async_remote_copy(
                src_ref=hloc_ref, dst_ref=hin_ref,
                send_sem=send_sem, recv_sem=recv_sem,
                device_id=((my + 1) % N_DEV,),
                device_id_type=pl.DeviceIdType.MESH,
            )
            rdma.wait_recv()
            h_in = hin_ref[...]
            for c in range(NC):
                sl = pl.ds(c * L, L)
                base = jnp.exp(jnp.float32(c * L + 1) * at)
                g = h_in * base[None]
                corr = (g[:, None] * powA[None]
                        * ce_ref[:, sl, :, :]).sum(axis=2)
                out_ref[:, sl, :] = out_ref[:, sl, :] + corr

    out = pl.pallas_call(
        body,
        out_shape=jax.ShapeDtypeStruct((Bsz, Sl, D), jnp.float32),
        in_specs=[pl.BlockSpec(memory_space=pltpu.VMEM)] * 4,
        out_specs=pl.BlockSpec(memory_space=pltpu.VMEM),
        scratch_shapes=[
            pltpu.VMEM((Bsz, N, D), jnp.float32),
            pltpu.VMEM((Bsz, N, D), jnp.float32),
            pltpu.SemaphoreType.DMA,
            pltpu.SemaphoreType.DMA,
        ],
        compiler_params=pltpu.CompilerParams(collective_id=0),
    )(x_e, AT, B_e, C_e)
    return out


# baseline (device time: 28881 ns/iter reference)
import jax
import jax.numpy as jnp
from jax import lax
from jax.experimental import pallas as pl
from jax.experimental.pallas import tpu as pltpu

N_DEV = 4
L = 32


def kernel(x, A, B, C):
    Bsz, Sl, D = x.shape
    N = A.shape[1]
    NC = Sl // L

    AT = A.T
    x_e = x[:, :, None, :]
    B_e = B[:, :, :, None]
    C_e = C[:, :, :, None]

    def body(xe_ref, at_ref, be_ref, ce_ref, out_ref,
             hloc_ref, hin_ref, send_sem, recv_sem):
        my = lax.axis_index("i")

        barrier = pltpu.get_barrier_semaphore()
        for k in range(1, N_DEV):
            pl.semaphore_signal(
                barrier, inc=1,
                device_id=((my + k) % N_DEV,),
                device_id_type=pl.DeviceIdType.MESH,
            )
        pl.semaphore_wait(barrier, N_DEV - 1)

        at = at_ref[:, :]
        tau = lax.broadcasted_iota(jnp.int32, (L, N, D), 0).astype(jnp.float32)
        powA = jnp.exp(tau * at[None])
        powAinv = jnp.exp(-tau * at[None])
        dA1 = jnp.exp(at)

        h = jnp.zeros((Bsz, N, D), jnp.float32)
        for c in range(NC):
            sl = pl.ds(c * L, L)
            xb = xe_ref[:, sl, :, :] * be_ref[:, sl, :, :]
            q = xb * powAinv[None]
            csum = q
            sh = 1
            while sh < L:
                csum = csum + jnp.concatenate(
                    [jnp.zeros((Bsz, sh, N, D), jnp.float32),
                     csum[:, : L - sh]], axis=1)
                sh *= 2
            h_all = powA[None] * (csum + (dA1[None] * h)[:, None])
            out_ref[:, sl, :] = (h_all * ce_ref[:, sl, :, :]).sum(
                axis=2, keepdims=False)
            h = h_all[:, L - 1]
        hloc_ref[...] = h

        @pl.when(my < N_DEV - 1)
        def _send():
            rdma = pltpu.make_async_remote_copy(
                src_ref=hloc_ref, dst_ref=hin_ref,
                send_sem=send_sem, recv_sem=recv_sem,
                device_id=((my + 1) % N_DEV,),
                device_id_type=pl.DeviceIdType.MESH,
            )
            rdma.start()
            rdma.wait_send()

        @pl.when(my > 0)
        def _recv_and_correct():
            rdma = pltpu.make_async_remote_copy(
                src_ref=hloc_ref, dst_ref=hin_ref,
                send_sem=send_sem, recv_sem=recv_sem,
                device_id=((my + 1) % N_DEV,),
                device_id_type=pl.DeviceIdType.MESH,
            )
            rdma.wait_recv()
            h_in = hin_ref[...]
            for c in range(NC):
                sl = pl.ds(c * L, L)
                base = jnp.exp(jnp.float32(c * L + 1) * at)
                g = h_in * base[None]
                corr = (g[:, None] * powA[None]
                        * ce_ref[:, sl, :, :]).sum(axis=2)
                out_ref[:, sl, :] = out_ref[:, sl, :] + corr

    out = pl.pallas_call(
        body,
        out_shape=jax.ShapeDtypeStruct((Bsz, Sl, D), jnp.float32),
        in_specs=[pl.BlockSpec(memory_space=pltpu.VMEM)] * 4,
        out_specs=pl.BlockSpec(memory_space=pltpu.VMEM),
        scratch_shapes=[
            pltpu.VMEM((Bsz, N, D), jnp.float32),
            pltpu.VMEM((Bsz, N, D), jnp.float32),
            pltpu.SemaphoreType.DMA,
            pltpu.SemaphoreType.DMA,
        ],
        compiler_params=pltpu.CompilerParams(collective_id=0),
    )(x_e, AT, B_e, C_e)
    return out


# device time: 23876 ns/iter; 1.2096x vs baseline; 1.2096x over previous
import jax
import jax.numpy as jnp
from jax import lax
from jax.experimental import pallas as pl
from jax.experimental.pallas import tpu as pltpu

N_DEV = 4
L = 32


def kernel(x, A, B, C):
    Bsz, Sl, D = x.shape
    N = A.shape[1]
    NC = Sl // L

    AT = A.T
    x_e = x[:, :, None, :]
    B_e = B[:, :, :, None]
    C_e = C[:, :, :, None]

    def body(xe_ref, at_ref, be_ref, ce_ref, out_ref,
             hloc_ref, hin_ref, send_sem, recv_sem):
        my = lax.axis_index("i")

        barrier = pltpu.get_barrier_semaphore()
        for k in range(1, N_DEV):
            pl.semaphore_signal(
                barrier, inc=1,
                device_id=((my + k) % N_DEV,),
                device_id_type=pl.DeviceIdType.MESH,
            )
        pl.semaphore_wait(barrier, N_DEV - 1)

        at = at_ref[:, :]
        dA1 = jnp.exp(at)[None, None]

        h = jnp.zeros((Bsz, 1, N, D), jnp.float32)
        h_chunks = []
        for c in range(NC):
            hs = []
            for t in range(c * L, (c + 1) * L):
                b_t = xe_ref[:, t:t + 1, :, :] * be_ref[:, t:t + 1, :, :]
                h = h * dA1 + b_t
                hs.append(h)
            h_chunks.append(jnp.concatenate(hs, axis=1))
        hloc_ref[...] = h[:, 0]

        send_rdma = pltpu.make_async_remote_copy(
            src_ref=hloc_ref, dst_ref=hin_ref,
            send_sem=send_sem, recv_sem=recv_sem,
            device_id=((my + 1) % N_DEV,),
            device_id_type=pl.DeviceIdType.MESH,
        )

        @pl.when(my < N_DEV - 1)
        def _send():
            send_rdma.start()

        for c in range(NC):
            sl = pl.ds(c * L, L)
            out_ref[:, sl, :] = (h_chunks[c] * ce_ref[:, sl, :, :]).sum(axis=2)

        @pl.when(my > 0)
        def _recv_and_correct():
            send_rdma.wait_recv()
            h_in = hin_ref[...]
            tau = lax.broadcasted_iota(
                jnp.int32, (L, N, D), 0).astype(jnp.float32)
            powA = jnp.exp(tau * at[None])
            for c in range(NC):
                sl = pl.ds(c * L, L)
                base = jnp.exp(jnp.float32(c * L + 1) * at)
                g = h_in * base[None]
                corr = (g[:, None] * powA[None]
                        * ce_ref[:, sl, :, :]).sum(axis=2)
                out_ref[:, sl, :] = out_ref[:, sl, :] + corr

        @pl.when(my < N_DEV - 1)
        def _wait_send():
            send_rdma.wait_send()

    out = pl.pallas_call(
        body,
        out_shape=jax.ShapeDtypeStruct((Bsz, Sl, D), jnp.float32),
        in_specs=[pl.BlockSpec(memory_space=pltpu.VMEM)] * 4,
        out_specs=pl.BlockSpec(memory_space=pltpu.VMEM),
        scratch_shapes=[
            pltpu.VMEM((Bsz, N, D), jnp.float32),
            pltpu.VMEM((Bsz, N, D), jnp.float32),
            pltpu.SemaphoreType.DMA,
            pltpu.SemaphoreType.DMA,
        ],
        compiler_params=pltpu.CompilerParams(collective_id=0),
    )(x_e, AT, B_e, C_e)
    return out


# device time: 13425 ns/iter; 2.1513x vs baseline; 1.7785x over previous
import jax
import jax.numpy as jnp
from jax import lax
from jax.experimental import pallas as pl
from jax.experimental.pallas import tpu as pltpu

N_DEV = 4
L = 32


def kernel(x, A, B, C):
    Bsz, Sl, D = x.shape
    N = A.shape[1]
    NC = Sl // L

    AT = jnp.transpose(A, (1, 0))
    CT = jnp.transpose(C, (0, 2, 1))

    def body(x_ref, at_ref, b_ref, ct_ref, out_ref,
             hloc_ref, hin_ref, send_sem, recv_sem):
        my = lax.axis_index("i")

        barrier = pltpu.get_barrier_semaphore()
        for k in (1, N_DEV - 1):
            pl.semaphore_signal(
                barrier, inc=1,
                device_id=((my + k) % N_DEV,),
                device_id_type=pl.DeviceIdType.MESH,
            )
        pl.semaphore_wait(barrier, 2)

        at = at_ref[:, :]
        dA1 = jnp.exp(at)[None, None]

        be_full = jnp.reshape(b_ref[...], (Bsz, Sl, N, 1))
        ce_full = jnp.reshape(
            jnp.transpose(ct_ref[...], (0, 2, 1)), (Bsz, Sl, N, 1))

        h = jnp.zeros((Bsz, 1, N, D), jnp.float32)
        h_chunks = []
        for c in range(NC):
            hs = []
            for t in range(c * L, (c + 1) * L):
                x_t = x_ref[:, t:t + 1, :][:, :, None, :]
                h = h * dA1 + x_t * be_full[:, t:t + 1]
                hs.append(h)
            h_chunks.append(jnp.concatenate(hs, axis=1))
        hloc_ref[...] = h[:, 0]

        send_rdma = pltpu.make_async_remote_copy(
            src_ref=hloc_ref, dst_ref=hin_ref,
            send_sem=send_sem, recv_sem=recv_sem,
            device_id=((my + 1) % N_DEV,),
            device_id_type=pl.DeviceIdType.MESH,
        )

        @pl.when(my < N_DEV - 1)
        def _send():
            send_rdma.start()

        for c in range(NC):
            out_ref[:, pl.ds(c * L, L), :] = (
                h_chunks[c] * ce_full[:, c * L:(c + 1) * L]).sum(axis=2)

        @pl.when(my > 0)
        def _recv_and_correct():
            send_rdma.wait_recv()
            h_in = hin_ref[...]
            tau = lax.broadcasted_iota(
                jnp.int32, (L, N, D), 0).astype(jnp.float32)
            powA = jnp.exp(tau * at[None])
            for c in range(NC):
                sl = pl.ds(c * L, L)
                base = jnp.exp(jnp.float32(c * L + 1) * at)
                g = h_in * base[None]
                corr = (g[:, None] * powA[None]
                        * ce_full[:, c * L:(c + 1) * L]).sum(axis=2)
                out_ref[:, sl, :] = out_ref[:, sl, :] + corr

        @pl.when(my < N_DEV - 1)
        def _wait_send():
            send_rdma.wait_send()

    out = pl.pallas_call(
        body,
        out_shape=jax.ShapeDtypeStruct((Bsz, Sl, D), jnp.float32),
        in_specs=[pl.BlockSpec(memory_space=pltpu.VMEM)] * 4,
        out_specs=pl.BlockSpec(memory_space=pltpu.VMEM),
        scratch_shapes=[
            pltpu.VMEM((Bsz, N, D), jnp.float32),
            pltpu.VMEM((Bsz, N, D), jnp.float32),
            pltpu.SemaphoreType.DMA,
            pltpu.SemaphoreType.DMA,
        ],
        compiler_params=pltpu.CompilerParams(collective_id=0),
    )(x, AT, B, CT)
    return out
